# baseline (speedup 1.0000x reference)
import functools

import jax
import jax.numpy as jnp
from jax import lax
from jax.experimental import pallas as pl
from jax.experimental.pallas import tpu as pltpu
from jax.experimental.pallas import tpu_sc as plsc

_NC = 2
_NS = 16
_NW = _NC * _NS
_B = 128


@functools.lru_cache(maxsize=None)
def _build_deg(n, eb, npad):
    bpw = eb // _NW
    rpt = npad // _NS
    k = 8

    mesh = plsc.VectorSubcoreMesh(core_axis_name="c", subcore_axis_name="s")

    @functools.partial(
        pl.kernel,
        out_type=jax.ShapeDtypeStruct((_NC * npad,), jnp.float32),
        mesh=mesh,
        scratch_types=[
            pltpu.VMEM_SHARED((npad,), jnp.float32),
            pltpu.VMEM((2, bpw, _B), jnp.int32),
            pltpu.VMEM((_B,), jnp.float32),
            pltpu.SemaphoreType.DMA,
        ],
    )
    def deg_kernel(ei_hbm, z_hbm, deg_hbm, deg_sh, idx3, ones_v, sem):
        c = lax.axis_index("c")
        s = lax.axis_index("s")
        wid = s * _NC + c
        ibase = pl.multiple_of(wid * bpw, 8)
        zbase = pl.multiple_of(s * rpt, rpt)

        one = jnp.ones((16,), jnp.float32)
        for i in range(_B // 16):
            ones_v[pl.ds(i * 16, 16)] = one
        pltpu.sync_copy(ei_hbm.at[:, pl.ds(ibase, bpw), :], idx3)
        pltpu.sync_copy(z_hbm.at[pl.ds(0, rpt)], deg_sh.at[pl.ds(zbase, rpt)])
        plsc.subcore_barrier()

        def grp(g, carry):
            for j in range(k):
                pltpu.async_copy(
                    ones_v, deg_sh.at[idx3.at[0, g * k + j]], sem, add=True)
            for j in range(k):
                pltpu.make_async_copy(
                    ones_v, deg_sh.at[idx3.at[0, g * k + j]], sem).wait()
            return carry

        lax.fori_loop(0, bpw // k, grp, 0)

        plsc.subcore_barrier()
        pltpu.sync_copy(deg_sh.at[pl.ds(zbase, rpt)],
                        deg_hbm.at[pl.ds(pl.multiple_of(c * npad + s * rpt, rpt), rpt)])

    return deg_kernel


@functools.lru_cache(maxsize=None)
def _build_agg(n, d, eb, npadr):
    bpw = eb // _NW
    npr = npadr // _NS

    mesh = plsc.VectorSubcoreMesh(core_axis_name="c", subcore_axis_name="s")

    nsec = 5
    sps = bpw // nsec
    hb = _B // 2

    @functools.partial(
        pl.kernel,
        out_type=jax.ShapeDtypeStruct((_NC, npadr, d), jnp.float32),
        mesh=mesh,
        scratch_types=[
            pltpu.VMEM_SHARED((npadr, d), jnp.float32),
            pltpu.VMEM((2, sps, 2, hb), jnp.int32),
            pltpu.VMEM((_B, d), jnp.float32),
            pltpu.VMEM((_B, d), jnp.float32),
            pltpu.SemaphoreType.DMA,
            pltpu.SemaphoreType.DMA,
            pltpu.SemaphoreType.DMA,
            pltpu.SemaphoreType.DMA,
        ],
    )
    def agg_kernel(h_hbm, ei_hbm, z_hbm, agg_hbm, agg_sh, idx4, r0, r1,
                   g0, g1, s0, s1):
        c = lax.axis_index("c")
        s = lax.axis_index("s")
        wid = s * _NC + c
        ibase = pl.multiple_of(wid * bpw, 8)
        rbase = pl.multiple_of(s * npr, 8)
        rbuf = (r0, r1)
        gsem = (g0, g1)
        ssem = (s0, s1)

        def gath(i, j):
            for hf in range(2):
                pltpu.async_copy(h_hbm.at[idx4.at[0, i, hf]],
                                 rbuf[j].at[pl.ds(hf * hb, hb)], gsem[j])

        def gath_wait(i, j):
            for hf in range(2):
                pltpu.make_async_copy(h_hbm.at[idx4.at[0, i, hf]],
                                      rbuf[j].at[pl.ds(hf * hb, hb)],
                                      gsem[j]).wait()

        def scat(i, j):
            for hf in range(2):
                pltpu.async_copy(rbuf[j].at[pl.ds(hf * hb, hb)],
                                 agg_sh.at[idx4.at[1, i, hf]], ssem[j],
                                 add=True)
            for hf in range(2):
                pltpu.make_async_copy(rbuf[j].at[pl.ds(hf * hb, hb)],
                                      agg_sh.at[idx4.at[1, i, hf]],
                                      ssem[j]).wait()

        pltpu.sync_copy(z_hbm.at[pl.ds(0, npr)], agg_sh.at[pl.ds(rbase, npr)])
        plsc.subcore_barrier()

        for sec in range(nsec):
            sbase = pl.multiple_of(ibase + sec * sps, 8)
            pltpu.sync_copy(ei_hbm.at[:, pl.ds(sbase, sps), :, :], idx4)
            gath(0, 0)
            gath(1, 1)

            def pair(t, carry):
                for j in range(2):
                    i = 2 * t + j
                    gath_wait(i, j)
                    scat(i, j)
                    gath(i + 2, j)
                return carry

            lax.fori_loop(0, sps // 2 - 1, pair, 0)
            for j in range(2):
                i = sps - 2 + j
                gath_wait(i, j)
                scat(i, j)

        plsc.subcore_barrier()
        pltpu.sync_copy(agg_sh.at[pl.ds(rbase, npr)],
                        agg_hbm.at[c, pl.ds(rbase, npr)])

    return agg_kernel


def _make_mm_body(n, npadr):
    def _mm_body(feat_ref, w_ref, deg_ref, h_ref):
        deg = deg_ref[0, :n] + deg_ref[1, :n]
        norm = lax.rsqrt(jnp.maximum(deg, 1.0))
        h = lax.dot_general(feat_ref[...], w_ref[...],
                            (((1,), (0,)), ((), ())),
                            precision=lax.Precision.HIGHEST,
                            preferred_element_type=jnp.float32)
        h_ref[:n] = h * norm
        h_ref[n:] = jnp.zeros((npadr - n, h.shape[1]), jnp.float32)
    return _mm_body


def _make_fin_body(n):
    def _fin_body(aggp_ref, deg_ref, b_ref, out_ref):
        agg = aggp_ref[0, :n] + aggp_ref[1, :n]
        norm = lax.rsqrt(jnp.maximum(deg_ref[0, :n] + deg_ref[1, :n], 1.0))
        out_ref[...] = agg * norm + b_ref[...]
    return _fin_body


def kernel(feat, edge_index, weight, bias):
    n, d = feat.shape
    e = edge_index.shape[1]
    npad = ((n + _NS * 16 - 1) // (_NS * 16)) * (_NS * 16)
    npadr = _NS * (((n + _NS * 8 - 1) // (_NS * 8)) * 8)

    epb = _NW * _B * 8
    epad = ((e + epb - 1) // epb) * epb
    if epad > e:
        pr = jnp.arange(epad - e, dtype=jnp.int32) % (npadr - n) + n
        ei = jnp.concatenate([edge_index, jnp.stack([pr, pr])], axis=1)
    else:
        ei = edge_index
    ei3 = ei.reshape(2, epad // _B, _B)
    eb = epad // _B

    zeros_hbm = jnp.zeros((npadr, d), jnp.float32)
    zeros1_hbm = jnp.zeros((npad,), jnp.float32)

    degf = _build_deg(n, eb, npad)(ei3, zeros1_hbm)
    degs = degf.reshape(_NC, npad, 1)

    h = pl.pallas_call(
        _make_mm_body(n, npadr),
        out_shape=jax.ShapeDtypeStruct((npadr, d), jnp.float32),
    )(feat, weight, degs)

    ei4 = ei3.reshape(2, eb, 2, _B // 2)
    aggp = _build_agg(n, d, eb, npadr)(h, ei4, zeros_hbm)

    out = pl.pallas_call(
        _make_fin_body(n),
        out_shape=jax.ShapeDtypeStruct((n, d), jnp.float32),
    )(aggp, degs, bias.reshape(1, d))
    return out


# speedup vs baseline: 1.1005x; 1.1005x over previous
import functools

import numpy as np

import jax
import jax.numpy as jnp
from jax import lax
from jax.experimental import pallas as pl
from jax.experimental.pallas import tpu as pltpu
from jax.experimental.pallas import tpu_sc as plsc

_NC = 2
_NS = 16
_NW = _NC * _NS
_B = 128


@functools.lru_cache(maxsize=None)
def _build_deg(n, eb, npad):
    bpw = eb // _NW
    rpt = npad // _NS
    k = 8

    mesh = plsc.VectorSubcoreMesh(core_axis_name="c", subcore_axis_name="s")

    @functools.partial(
        pl.kernel,
        out_type=jax.ShapeDtypeStruct((_NC * npad,), jnp.float32),
        mesh=mesh,
        scratch_types=[
            pltpu.VMEM_SHARED((npad,), jnp.float32),
            pltpu.VMEM((2, bpw, _B), jnp.int32),
            pltpu.VMEM((_B,), jnp.float32),
            pltpu.SemaphoreType.DMA,
        ],
    )
    def deg_kernel(ei_hbm, z_hbm, deg_hbm, deg_sh, idx3, ones_v, sem):
        c = lax.axis_index("c")
        s = lax.axis_index("s")
        wid = s * _NC + c
        ibase = pl.multiple_of(wid * bpw, 8)
        zbase = pl.multiple_of(s * rpt, rpt)

        one = jnp.ones((16,), jnp.float32)
        for i in range(_B // 16):
            ones_v[pl.ds(i * 16, 16)] = one
        pltpu.sync_copy(ei_hbm.at[:, pl.ds(ibase, bpw), :], idx3)
        pltpu.sync_copy(z_hbm.at[pl.ds(0, rpt)], deg_sh.at[pl.ds(zbase, rpt)])
        plsc.subcore_barrier()

        def grp(g, carry):
            for j in range(k):
                pltpu.async_copy(
                    ones_v, deg_sh.at[idx3.at[0, g * k + j]], sem, add=True)
            for j in range(k):
                pltpu.make_async_copy(
                    ones_v, deg_sh.at[idx3.at[0, g * k + j]], sem).wait()
            return carry

        lax.fori_loop(0, bpw // k, grp, 0)

        plsc.subcore_barrier()
        pltpu.sync_copy(deg_sh.at[pl.ds(zbase, rpt)],
                        deg_hbm.at[pl.ds(pl.multiple_of(c * npad + s * rpt, rpt), rpt)])

    return deg_kernel


@functools.lru_cache(maxsize=None)
def _build_agg(n, d, eb, npadr):
    bpw = eb // _NW
    npr = npadr // _NS

    mesh = plsc.VectorSubcoreMesh(core_axis_name="c", subcore_axis_name="s")

    nsec = 2
    sps = bpw // nsec

    @functools.partial(
        pl.kernel,
        out_type=jax.ShapeDtypeStruct((_NC, npadr, d), jnp.float32),
        mesh=mesh,
        scratch_types=[
            pltpu.VMEM_SHARED((npadr, d), jnp.float32),
            pltpu.VMEM((2, sps, _B), jnp.int32),
            pltpu.VMEM((_B, d), jnp.float32),
            pltpu.VMEM((_B, d), jnp.float32),
            pltpu.SemaphoreType.DMA,
            pltpu.SemaphoreType.DMA,
        ],
    )
    def agg_kernel(h_hbm, ei_hbm, z_hbm, agg_hbm, agg_sh, idx3, r0, r1,
                   sem0, sem1):
        c = lax.axis_index("c")
        s = lax.axis_index("s")
        wid = s * _NC + c
        ibase = pl.multiple_of(wid * bpw, 8)
        rbase = pl.multiple_of(s * npr, 8)
        rbuf = (r0, r1)
        sems = (sem0, sem1)

        pltpu.sync_copy(z_hbm.at[pl.ds(0, npr)], agg_sh.at[pl.ds(rbase, npr)])
        plsc.subcore_barrier()

        for sec in range(nsec):
            sbase = pl.multiple_of(ibase + sec * sps, 8)
            pltpu.sync_copy(ei_hbm.at[:, pl.ds(sbase, sps), :], idx3)
            pltpu.async_copy(h_hbm.at[idx3.at[0, 0]], r0, sem0)
            pltpu.async_copy(h_hbm.at[idx3.at[0, 1]], r1, sem1)

            def pair(t, carry):
                for j in range(2):
                    i = 2 * t + j
                    pltpu.make_async_copy(
                        h_hbm.at[idx3.at[0, i]], rbuf[j], sems[j]).wait()
                    pltpu.sync_copy(rbuf[j], agg_sh.at[idx3.at[1, i]],
                                    add=True)
                    pltpu.async_copy(
                        h_hbm.at[idx3.at[0, i + 2]], rbuf[j], sems[j])
                return carry

            lax.fori_loop(0, sps // 2 - 1, pair, 0)
            for j in range(2):
                i = sps - 2 + j
                pltpu.make_async_copy(
                    h_hbm.at[idx3.at[0, i]], rbuf[j], sems[j]).wait()
                pltpu.sync_copy(rbuf[j], agg_sh.at[idx3.at[1, i]], add=True)

        plsc.subcore_barrier()
        pltpu.sync_copy(agg_sh.at[pl.ds(rbase, npr)],
                        agg_hbm.at[c, pl.ds(rbase, npr)])

    return agg_kernel


def _make_mm_body(n, npadr):
    def _mm_body(feat_ref, w_ref, deg_ref, h_ref):
        deg = deg_ref[0, :n] + deg_ref[1, :n]
        norm = lax.rsqrt(jnp.maximum(deg, 1.0))
        h = lax.dot_general(feat_ref[...], w_ref[...],
                            (((1,), (0,)), ((), ())),
                            precision=lax.Precision.DEFAULT,
                            preferred_element_type=jnp.float32)
        h_ref[:n] = h * norm
        h_ref[n:] = jnp.zeros((npadr - n, h.shape[1]), jnp.float32)
    return _mm_body


def _make_fin_body(n):
    def _fin_body(aggp_ref, deg_ref, b_ref, out_ref):
        agg = aggp_ref[0, :n] + aggp_ref[1, :n]
        norm = lax.rsqrt(jnp.maximum(deg_ref[0, :n] + deg_ref[1, :n], 1.0))
        out_ref[...] = agg * norm + b_ref[...]
    return _fin_body


def kernel(feat, edge_index, weight, bias):
    n, d = feat.shape
    e = edge_index.shape[1]
    npad = ((n + _NS * 16 - 1) // (_NS * 16)) * (_NS * 16)
    npadr = _NS * (((n + _NS * 8 - 1) // (_NS * 8)) * 8)

    epb = _NW * _B * 8
    epad = ((e + epb - 1) // epb) * epb
    assert e % _B == 0
    if epad > e:
        pr = np.arange(epad - e, dtype=np.int32) % (npadr - n) + n
        pad3 = np.broadcast_to(pr, (2, epad - e)).reshape(2, (epad - e) // _B, _B)
        ei3 = jnp.concatenate(
            [edge_index.reshape(2, e // _B, _B), jnp.asarray(pad3)], axis=1)
    else:
        ei3 = edge_index.reshape(2, e // _B, _B)
    eb = epad // _B

    zeros_hbm = np.zeros((npadr, d), np.float32)
    zeros1_hbm = np.zeros((npad,), np.float32)

    degf = _build_deg(n, eb, npad)(ei3, zeros1_hbm)
    degs = degf.reshape(_NC, npad, 1)

    h = pl.pallas_call(
        _make_mm_body(n, npadr),
        out_shape=jax.ShapeDtypeStruct((npadr, d), jnp.float32),
    )(feat, weight, degs)

    aggp = _build_agg(n, d, eb, npadr)(h, ei3, zeros_hbm)

    out = pl.pallas_call(
        _make_fin_body(n),
        out_shape=jax.ShapeDtypeStruct((n, d), jnp.float32),
    )(aggp, degs, bias.reshape(1, d))
    return out


# speedup vs baseline: 1.2360x; 1.1232x over previous
import functools

import numpy as np

import jax
import jax.numpy as jnp
from jax import lax
from jax.experimental import pallas as pl
from jax.experimental.pallas import tpu as pltpu
from jax.experimental.pallas import tpu_sc as plsc

_NC = 2
_NS = 16
_NW = _NC * _NS
_B = 128


@functools.lru_cache(maxsize=None)
def _build_deg(n, eb, npad):
    bpw = eb // _NW
    rpt = npad // _NS
    k = 8

    mesh = plsc.VectorSubcoreMesh(core_axis_name="c", subcore_axis_name="s")

    @functools.partial(
        pl.kernel,
        out_type=jax.ShapeDtypeStruct((_NC * npad,), jnp.float32),
        mesh=mesh,
        scratch_types=[
            pltpu.VMEM_SHARED((npad,), jnp.float32),
            pltpu.VMEM((2, bpw, _B), jnp.int32),
            pltpu.VMEM((_B,), jnp.float32),
            pltpu.SemaphoreType.DMA,
        ],
    )
    def deg_kernel(ei_hbm, z_hbm, deg_hbm, deg_sh, idx3, ones_v, sem):
        c = lax.axis_index("c")
        s = lax.axis_index("s")
        wid = s * _NC + c
        ibase = pl.multiple_of(wid * bpw, 8)
        zbase = pl.multiple_of(s * rpt, rpt)

        one = jnp.ones((16,), jnp.float32)
        for i in range(_B // 16):
            ones_v[pl.ds(i * 16, 16)] = one
        pltpu.sync_copy(ei_hbm.at[:, pl.ds(ibase, bpw), :], idx3)
        pltpu.sync_copy(z_hbm.at[pl.ds(0, rpt)], deg_sh.at[pl.ds(zbase, rpt)])
        plsc.subcore_barrier()

        def grp(g, carry):
            for j in range(k):
                pltpu.async_copy(
                    ones_v, deg_sh.at[idx3.at[0, g * k + j]], sem, add=True)
            for j in range(k):
                pltpu.make_async_copy(
                    ones_v, deg_sh.at[idx3.at[0, g * k + j]], sem).wait()
            return carry

        lax.fori_loop(0, bpw // k, grp, 0)

        plsc.subcore_barrier()
        pltpu.sync_copy(deg_sh.at[pl.ds(zbase, rpt)],
                        deg_hbm.at[pl.ds(pl.multiple_of(c * npad + s * rpt, rpt), rpt)])

    return deg_kernel


@functools.lru_cache(maxsize=None)
def _build_agg(n, d, eb, npadr):
    bpw = eb // _NW
    npr = npadr // _NS

    mesh = plsc.VectorSubcoreMesh(core_axis_name="c", subcore_axis_name="s")

    nsec = 2
    sps = bpw // nsec

    @functools.partial(
        pl.kernel,
        out_type=jax.ShapeDtypeStruct((_NC, npadr, d), jnp.float32),
        mesh=mesh,
        scratch_types=[
            pltpu.VMEM_SHARED((npadr, d), jnp.float32),
            pltpu.VMEM((2, sps, _B), jnp.int32),
            pltpu.VMEM((_B, d), jnp.float32),
            pltpu.VMEM((_B, d), jnp.float32),
            pltpu.SemaphoreType.DMA,
            pltpu.SemaphoreType.DMA,
        ],
    )
    def agg_kernel(h_hbm, ei_hbm, agg_hbm, agg_sh, idx3, r0, r1,
                   sem0, sem1):
        c = lax.axis_index("c")
        s = lax.axis_index("s")
        wid = s * _NC + c
        ibase = pl.multiple_of(wid * bpw, 8)
        rbase = pl.multiple_of(s * npr, 8)
        rbuf = (r0, r1)
        sems = (sem0, sem1)

        pltpu.sync_copy(h_hbm.at[pl.ds(npadr, npr)],
                        agg_sh.at[pl.ds(rbase, npr)])
        plsc.subcore_barrier()

        for sec in range(nsec):
            sbase = pl.multiple_of(ibase + sec * sps, 8)
            pltpu.sync_copy(ei_hbm.at[:, pl.ds(sbase, sps), :], idx3)
            pltpu.async_copy(h_hbm.at[idx3.at[0, 0]], r0, sem0)
            pltpu.async_copy(h_hbm.at[idx3.at[0, 1]], r1, sem1)

            def pair(t, carry):
                for j in range(2):
                    i = 2 * t + j
                    pltpu.make_async_copy(
                        h_hbm.at[idx3.at[0, i]], rbuf[j], sems[j]).wait()
                    pltpu.sync_copy(rbuf[j], agg_sh.at[idx3.at[1, i]],
                                    add=True)
                    pltpu.async_copy(
                        h_hbm.at[idx3.at[0, i + 2]], rbuf[j], sems[j])
                return carry

            lax.fori_loop(0, sps // 2 - 1, pair, 0)
            for j in range(2):
                i = sps - 2 + j
                pltpu.make_async_copy(
                    h_hbm.at[idx3.at[0, i]], rbuf[j], sems[j]).wait()
                pltpu.sync_copy(rbuf[j], agg_sh.at[idx3.at[1, i]], add=True)

        plsc.subcore_barrier()
        pltpu.sync_copy(agg_sh.at[pl.ds(rbase, npr)],
                        agg_hbm.at[c, pl.ds(rbase, npr)])

    return agg_kernel


def _make_mm_body(n, hz):
    def _mm_body(feat_ref, w_ref, deg_ref, h_ref):
        deg = deg_ref[0, :n] + deg_ref[1, :n]
        norm = lax.rsqrt(jnp.maximum(deg, 1.0))[:, None]
        h = lax.dot_general(feat_ref[...], w_ref[...],
                            (((1,), (0,)), ((), ())),
                            precision=lax.Precision.DEFAULT,
                            preferred_element_type=jnp.float32)
        h_ref[:n] = h * norm
        h_ref[n:] = jnp.zeros((hz - n, h.shape[1]), jnp.float32)
    return _mm_body


def _make_fin_body(n):
    def _fin_body(aggp_ref, deg_ref, b_ref, out_ref):
        agg = aggp_ref[0, :n] + aggp_ref[1, :n]
        deg = deg_ref[0, :n] + deg_ref[1, :n]
        norm = lax.rsqrt(jnp.maximum(deg, 1.0))[:, None]
        out_ref[...] = agg * norm + b_ref[...]
    return _fin_body


def kernel(feat, edge_index, weight, bias):
    n, d = feat.shape
    e = edge_index.shape[1]
    npad = ((n + _NS * 16 - 1) // (_NS * 16)) * (_NS * 16)
    npadr = _NS * (((n + _NS * 8 - 1) // (_NS * 8)) * 8)

    epb = _NW * _B * 8
    epad = ((e + epb - 1) // epb) * epb
    assert e % _B == 0
    if epad > e:
        pr = np.arange(epad - e, dtype=np.int32) % (npadr - n) + n
        pad3 = np.broadcast_to(pr, (2, epad - e)).reshape(2, (epad - e) // _B, _B)
        ei3 = jnp.concatenate(
            [edge_index.reshape(2, e // _B, _B), jnp.asarray(pad3)], axis=1)
    else:
        ei3 = edge_index.reshape(2, e // _B, _B)
    eb = epad // _B

    zeros1_hbm = np.zeros((npad,), np.float32)

    degf = _build_deg(n, eb, npad)(ei3, zeros1_hbm)
    degs = degf.reshape(_NC, npad)

    hz = npadr + 640
    h = pl.pallas_call(
        _make_mm_body(n, hz),
        out_shape=jax.ShapeDtypeStruct((hz, d), jnp.float32),
    )(feat, weight, degs)

    aggp = _build_agg(n, d, eb, npadr)(h, ei3)

    out = pl.pallas_call(
        _make_fin_body(n),
        out_shape=jax.ShapeDtypeStruct((n, d), jnp.float32),
    )(aggp, degs, bias.reshape(1, d))
    return out


# speedup vs baseline: 1.2377x; 1.0014x over previous
import functools

import numpy as np

import jax
import jax.numpy as jnp
from jax import lax
from jax.experimental import pallas as pl
from jax.experimental.pallas import tpu as pltpu
from jax.experimental.pallas import tpu_sc as plsc

_NC = 2
_NS = 16
_NW = _NC * _NS
_B = 128


@functools.lru_cache(maxsize=None)
def _build_deg(n, eb, npad):
    bpw = eb // _NW
    rpt = npad // _NS
    k = 16

    mesh = plsc.VectorSubcoreMesh(core_axis_name="c", subcore_axis_name="s")

    @functools.partial(
        pl.kernel,
        out_type=jax.ShapeDtypeStruct((_NC * npad,), jnp.float32),
        mesh=mesh,
        scratch_types=[
            pltpu.VMEM_SHARED((npad,), jnp.float32),
            pltpu.VMEM((2, bpw, _B), jnp.int32),
            pltpu.VMEM((_B,), jnp.float32),
            pltpu.SemaphoreType.DMA,
        ],
    )
    def deg_kernel(ei_hbm, z_hbm, deg_hbm, deg_sh, idx3, ones_v, sem):
        c = lax.axis_index("c")
        s = lax.axis_index("s")
        wid = s * _NC + c
        ibase = pl.multiple_of(wid * bpw, 8)
        zbase = pl.multiple_of(s * rpt, rpt)

        one = jnp.ones((16,), jnp.float32)
        for i in range(_B // 16):
            ones_v[pl.ds(i * 16, 16)] = one
        pltpu.sync_copy(ei_hbm.at[:, pl.ds(ibase, bpw), :], idx3)
        pltpu.sync_copy(z_hbm.at[pl.ds(0, rpt)], deg_sh.at[pl.ds(zbase, rpt)])
        plsc.subcore_barrier()

        def grp(g, carry):
            for j in range(k):
                pltpu.async_copy(
                    ones_v, deg_sh.at[idx3.at[0, g * k + j]], sem, add=True)
            for j in range(k):
                pltpu.make_async_copy(
                    ones_v, deg_sh.at[idx3.at[0, g * k + j]], sem).wait()
            return carry

        lax.fori_loop(0, bpw // k, grp, 0)

        plsc.subcore_barrier()
        pltpu.sync_copy(deg_sh.at[pl.ds(zbase, rpt)],
                        deg_hbm.at[pl.ds(pl.multiple_of(c * npad + s * rpt, rpt), rpt)])

    return deg_kernel


@functools.lru_cache(maxsize=None)
def _build_agg(n, d, eb, npadr):
    bpw = eb // _NW
    npr = npadr // _NS

    mesh = plsc.VectorSubcoreMesh(core_axis_name="c", subcore_axis_name="s")

    nsec = 2
    sps = bpw // nsec

    @functools.partial(
        pl.kernel,
        out_type=jax.ShapeDtypeStruct((_NC, npadr, d), jnp.float32),
        mesh=mesh,
        scratch_types=[
            pltpu.VMEM_SHARED((npadr, d), jnp.float32),
            pltpu.VMEM((2, sps, _B), jnp.int32),
            pltpu.VMEM((_B, d), jnp.float32),
            pltpu.VMEM((_B, d), jnp.float32),
            pltpu.SemaphoreType.DMA,
            pltpu.SemaphoreType.DMA,
        ],
    )
    def agg_kernel(h_hbm, ei_hbm, agg_hbm, agg_sh, idx3, r0, r1,
                   sem0, sem1):
        c = lax.axis_index("c")
        s = lax.axis_index("s")
        wid = s * _NC + c
        ibase = pl.multiple_of(wid * bpw, 8)
        rbase = pl.multiple_of(s * npr, 8)
        rbuf = (r0, r1)
        sems = (sem0, sem1)

        pltpu.sync_copy(h_hbm.at[pl.ds(npadr, npr)],
                        agg_sh.at[pl.ds(rbase, npr)])
        plsc.subcore_barrier()

        for sec in range(nsec):
            sbase = pl.multiple_of(ibase + sec * sps, 8)
            pltpu.sync_copy(ei_hbm.at[:, pl.ds(sbase, sps), :], idx3)
            pltpu.async_copy(h_hbm.at[idx3.at[0, 0]], r0, sem0)
            pltpu.async_copy(h_hbm.at[idx3.at[0, 1]], r1, sem1)

            def pair(t, carry):
                for j in range(2):
                    i = 2 * t + j
                    pltpu.make_async_copy(
                        h_hbm.at[idx3.at[0, i]], rbuf[j], sems[j]).wait()
                    pltpu.sync_copy(rbuf[j], agg_sh.at[idx3.at[1, i]],
                                    add=True)
                    pltpu.async_copy(
                        h_hbm.at[idx3.at[0, i + 2]], rbuf[j], sems[j])
                return carry

            lax.fori_loop(0, sps // 2 - 1, pair, 0)
            for j in range(2):
                i = sps - 2 + j
                pltpu.make_async_copy(
                    h_hbm.at[idx3.at[0, i]], rbuf[j], sems[j]).wait()
                pltpu.sync_copy(rbuf[j], agg_sh.at[idx3.at[1, i]], add=True)

        plsc.subcore_barrier()
        pltpu.sync_copy(agg_sh.at[pl.ds(rbase, npr)],
                        agg_hbm.at[c, pl.ds(rbase, npr)])

    return agg_kernel


def _make_mm_body(n, hz):
    def _mm_body(feat_ref, w_ref, deg_ref, h_ref):
        deg = deg_ref[0, :n] + deg_ref[1, :n]
        norm = lax.rsqrt(jnp.maximum(deg, 1.0))[:, None]
        h = lax.dot_general(feat_ref[...], w_ref[...],
                            (((1,), (0,)), ((), ())),
                            precision=lax.Precision.DEFAULT,
                            preferred_element_type=jnp.float32)
        h_ref[:n] = h * norm
        h_ref[n:] = jnp.zeros((hz - n, h.shape[1]), jnp.float32)
    return _mm_body


def _make_fin_body(n):
    def _fin_body(aggp_ref, deg_ref, b_ref, out_ref):
        agg = aggp_ref[0, :n] + aggp_ref[1, :n]
        deg = deg_ref[0, :n] + deg_ref[1, :n]
        norm = lax.rsqrt(jnp.maximum(deg, 1.0))[:, None]
        out_ref[...] = agg * norm + b_ref[...]
    return _fin_body


def kernel(feat, edge_index, weight, bias):
    n, d = feat.shape
    e = edge_index.shape[1]
    npad = ((n + _NS * 16 - 1) // (_NS * 16)) * (_NS * 16)
    npadr = _NS * (((n + _NS * 8 - 1) // (_NS * 8)) * 8)

    epb = _NW * _B * 8
    epad = ((e + epb - 1) // epb) * epb
    assert e % _B == 0
    if epad > e:
        pr = np.arange(epad - e, dtype=np.int32) % (npadr - n) + n
        pad3 = np.broadcast_to(pr, (2, epad - e)).reshape(2, (epad - e) // _B, _B)
        ei3 = jnp.concatenate(
            [edge_index.reshape(2, e // _B, _B), jnp.asarray(pad3)], axis=1)
    else:
        ei3 = edge_index.reshape(2, e // _B, _B)
    eb = epad // _B

    zeros1_hbm = np.zeros((npad,), np.float32)

    degf = _build_deg(n, eb, npad)(ei3, zeros1_hbm)
    degs = degf.reshape(_NC, npad)

    hz = npadr + 640
    h = pl.pallas_call(
        _make_mm_body(n, hz),
        out_shape=jax.ShapeDtypeStruct((hz, d), jnp.float32),
    )(feat, weight, degs)

    aggp = _build_agg(n, d, eb, npadr)(h, ei3)

    out = pl.pallas_call(
        _make_fin_body(n),
        out_shape=jax.ShapeDtypeStruct((n, d), jnp.float32),
    )(aggp, degs, bias.reshape(1, d))
    return out


# speedup vs baseline: 1.2500x; 1.0100x over previous
import functools

import numpy as np

import jax
import jax.numpy as jnp
from jax import lax
from jax.experimental import pallas as pl
from jax.experimental.pallas import tpu as pltpu
from jax.experimental.pallas import tpu_sc as plsc

_NC = 2
_NS = 16
_NW = _NC * _NS
_B = 128


@functools.lru_cache(maxsize=None)
def _build_deg(n, eb, npad):
    bpw = eb // _NW
    rpt = npad // _NS
    k = 16

    mesh = plsc.VectorSubcoreMesh(core_axis_name="c", subcore_axis_name="s")

    @functools.partial(
        pl.kernel,
        out_type=jax.ShapeDtypeStruct((_NC * npad,), jnp.float32),
        mesh=mesh,
        scratch_types=[
            pltpu.VMEM_SHARED((npad,), jnp.float32),
            pltpu.VMEM((2, bpw, _B), jnp.int32),
            pltpu.VMEM((_B,), jnp.float32),
            pltpu.SemaphoreType.DMA,
        ],
    )
    def deg_kernel(ei_hbm, z_hbm, deg_hbm, deg_sh, idx3, ones_v, sem):
        c = lax.axis_index("c")
        s = lax.axis_index("s")
        wid = s * _NC + c
        ibase = pl.multiple_of(wid * bpw, 8)
        zbase = pl.multiple_of(s * rpt, rpt)

        one = jnp.ones((16,), jnp.float32)
        for i in range(_B // 16):
            ones_v[pl.ds(i * 16, 16)] = one
        pltpu.sync_copy(ei_hbm.at[:, pl.ds(ibase, bpw), :], idx3)
        pltpu.sync_copy(z_hbm.at[pl.ds(0, rpt)], deg_sh.at[pl.ds(zbase, rpt)])
        plsc.subcore_barrier()

        def grp(g, carry):
            for j in range(k):
                pltpu.async_copy(
                    ones_v, deg_sh.at[idx3.at[0, g * k + j]], sem, add=True)
            for j in range(k):
                pltpu.make_async_copy(
                    ones_v, deg_sh.at[idx3.at[0, g * k + j]], sem).wait()
            return carry

        lax.fori_loop(0, bpw // k, grp, 0)

        plsc.subcore_barrier()
        pltpu.sync_copy(deg_sh.at[pl.ds(zbase, rpt)],
                        deg_hbm.at[pl.ds(pl.multiple_of(c * npad + s * rpt, rpt), rpt)])

    return deg_kernel


@functools.lru_cache(maxsize=None)
def _build_agg(n, d, eb, npadr):
    bpw = eb // _NW
    npr = npadr // _NS

    mesh = plsc.VectorSubcoreMesh(core_axis_name="c", subcore_axis_name="s")

    nsec = 2
    sps = bpw // nsec

    @functools.partial(
        pl.kernel,
        out_type=jax.ShapeDtypeStruct((_NC, npadr, d), jnp.float32),
        mesh=mesh,
        scratch_types=[
            pltpu.VMEM_SHARED((npadr, d), jnp.float32),
            pltpu.VMEM((2, sps, _B), jnp.int32),
            pltpu.VMEM((_B, d), jnp.float32),
            pltpu.VMEM((_B, d), jnp.float32),
            pltpu.SemaphoreType.DMA,
            pltpu.SemaphoreType.DMA,
        ],
    )
    def agg_kernel(h_hbm, ei_hbm, agg_hbm, agg_sh, idx3, r0, r1,
                   sem0, sem1):
        c = lax.axis_index("c")
        s = lax.axis_index("s")
        wid = s * _NC + c
        ibase = pl.multiple_of(wid * bpw, 8)
        rbase = pl.multiple_of(s * npr, 8)
        rbuf = (r0, r1)
        sems = (sem0, sem1)

        pltpu.sync_copy(ei_hbm.at[:, pl.ds(ibase, sps), :], idx3)
        pltpu.sync_copy(h_hbm.at[pl.ds(npadr, npr)],
                        agg_sh.at[pl.ds(rbase, npr)])
        plsc.subcore_barrier()

        for sec in range(nsec):
            sbase = pl.multiple_of(ibase + sec * sps, 8)
            if sec > 0:
                pltpu.sync_copy(ei_hbm.at[:, pl.ds(sbase, sps), :], idx3)
            pltpu.async_copy(h_hbm.at[idx3.at[0, 0]], r0, sem0)
            pltpu.async_copy(h_hbm.at[idx3.at[0, 1]], r1, sem1)

            def pair(t, carry):
                for j in range(2):
                    i = 2 * t + j
                    pltpu.make_async_copy(
                        h_hbm.at[idx3.at[0, i]], rbuf[j], sems[j]).wait()
                    pltpu.sync_copy(rbuf[j], agg_sh.at[idx3.at[1, i]],
                                    add=True)
                    pltpu.async_copy(
                        h_hbm.at[idx3.at[0, i + 2]], rbuf[j], sems[j])
                return carry

            lax.fori_loop(0, sps // 2 - 1, pair, 0)
            for j in range(2):
                i = sps - 2 + j
                pltpu.make_async_copy(
                    h_hbm.at[idx3.at[0, i]], rbuf[j], sems[j]).wait()
                pltpu.sync_copy(rbuf[j], agg_sh.at[idx3.at[1, i]], add=True)

        plsc.subcore_barrier()
        pltpu.sync_copy(agg_sh.at[pl.ds(rbase, npr)],
                        agg_hbm.at[c, pl.ds(rbase, npr)])

    return agg_kernel


def _make_mm_body(n, hz, npad):
    def _mm_body(feat_ref, w_ref, deg_ref, h_ref):
        deg = deg_ref[:n] + deg_ref[npad:npad + n]
        norm = lax.rsqrt(jnp.maximum(deg, 1.0))[:, None]
        h = lax.dot_general(feat_ref[...], w_ref[...],
                            (((1,), (0,)), ((), ())),
                            precision=lax.Precision.DEFAULT,
                            preferred_element_type=jnp.float32)
        h_ref[:n] = h * norm
        h_ref[n:] = jnp.zeros((hz - n, h.shape[1]), jnp.float32)
    return _mm_body


def _make_fin_body(n, npad):
    def _fin_body(aggp_ref, deg_ref, b_ref, out_ref):
        agg = aggp_ref[0, :n] + aggp_ref[1, :n]
        deg = deg_ref[:n] + deg_ref[npad:npad + n]
        norm = lax.rsqrt(jnp.maximum(deg, 1.0))[:, None]
        out_ref[...] = agg * norm + b_ref[...]
    return _fin_body


def kernel(feat, edge_index, weight, bias):
    n, d = feat.shape
    e = edge_index.shape[1]
    npad = ((n + _NS * 16 - 1) // (_NS * 16)) * (_NS * 16)
    npadr = _NS * (((n + _NS * 8 - 1) // (_NS * 8)) * 8)

    epb = _NW * _B * 8
    epad = ((e + epb - 1) // epb) * epb
    assert e % _B == 0
    if epad > e:
        pr = np.arange(epad - e, dtype=np.int32) % (npadr - n) + n
        pad3 = np.broadcast_to(pr, (2, epad - e)).reshape(2, (epad - e) // _B, _B)
        ei3 = jnp.concatenate(
            [edge_index.reshape(2, e // _B, _B), jnp.asarray(pad3)], axis=1)
    else:
        ei3 = edge_index.reshape(2, e // _B, _B)
    eb = epad // _B

    zeros1_hbm = np.zeros((npad,), np.float32)

    degf = _build_deg(n, eb, npad)(ei3, zeros1_hbm)

    hz = npadr + 640
    h = pl.pallas_call(
        _make_mm_body(n, hz, npad),
        out_shape=jax.ShapeDtypeStruct((hz, d), jnp.float32),
    )(feat, weight, degf)

    aggp = _build_agg(n, d, eb, npadr)(h, ei3)

    out = pl.pallas_call(
        _make_fin_body(n, npad),
        out_shape=jax.ShapeDtypeStruct((n, d), jnp.float32),
    )(aggp, degf, bias.reshape(1, d))
    return out
